# baseline (speedup 1.0000x reference)
import functools

import jax
import jax.numpy as jnp
from jax import lax
from jax.experimental import pallas as pl
from jax.experimental.pallas import tpu as pltpu
from jax.experimental.pallas import tpu_sc as plsc

NUM_EMB = 1000000
BATCH = 16384
QLEN = 50
LPAD = 56
DIM = 16
OUT = 128
BLP = BATCH * LPAD
G = 128
K = 8
CHUNK_ROWS = K * G
TBLK = 1024
WBLK = 8192
TGRID = -(-NUM_EMB // WBLK)
VPAD = TGRID * WBLK


def _relayout_body(x_ref, o_ref):
    x = x_ref[...]
    for j in range(WBLK // TBLK):
        xj = x[:, TBLK * j:TBLK * (j + 1)]
        xh = xj.reshape(DIM, 8, 128).swapaxes(0, 1).reshape(128, 128)
        o_ref[128 * j:128 * (j + 1), :] = xh.T


def _tc_relayout(t_t):
    return pl.pallas_call(
        _relayout_body,
        grid=(TGRID,),
        in_specs=[pl.BlockSpec((DIM, WBLK), lambda i: (0, i))],
        out_specs=pl.BlockSpec((WBLK // 8, 128), lambda i: (i, 0)),
        out_shape=jax.ShapeDtypeStruct((VPAD // 8, 128), jnp.float32),
    )(t_t)


def _permute_idx(v):
    return (v & ~(TBLK - 1)) + ((v & 127) << 3) + ((v >> 7) & 7)


def _sc_gather(q2, table):
    info = plsc.get_sparse_core_info()
    nc, ns = info.num_cores, info.num_subcores
    nw = nc * ns
    qrows_per_w = q2.shape[0] // nw
    chunks = qrows_per_w // K

    mesh = plsc.VectorSubcoreMesh(core_axis_name="c", subcore_axis_name="s")

    @functools.partial(
        pl.kernel,
        mesh=mesh,
        compiler_params=pltpu.CompilerParams(use_tc_tiling_on_sc=False),
        out_type=jax.ShapeDtypeStruct((BLP, DIM), jnp.float32),
        scratch_types=[
            pltpu.VMEM((2, K, G), jnp.int32),
            pltpu.VMEM((2, CHUNK_ROWS, DIM), jnp.float32),
            pltpu.SemaphoreType.DMA,
            pltpu.SemaphoreType.DMA,
        ],
    )
    def gk(q_hbm, t_hbm, out_hbm, idx_v, rows_v, sem0, sem1):
        sems = (sem0, sem1)
        wid = lax.axis_index("s") * nc + lax.axis_index("c")
        base = wid * qrows_per_w

        def fire(c, buf):
            pltpu.sync_copy(q_hbm.at[pl.ds(base + c * K, K)], idx_v.at[buf])
            for j in range(K):
                pltpu.async_copy(
                    t_hbm.at[idx_v.at[buf].at[j]],
                    rows_v.at[buf].at[pl.ds(j * G, G)],
                    sems[buf],
                )

        def drain_write(c, buf):
            pltpu.make_async_copy(
                t_hbm.at[pl.ds(0, CHUNK_ROWS)], rows_v.at[buf], sems[buf]
            ).wait()
            pltpu.sync_copy(
                rows_v.at[buf],
                out_hbm.at[pl.ds((base + c * K) * G, CHUNK_ROWS)],
            )

        fire(0, 0)

        @pl.loop(0, chunks - 2, step=2)
        def _pair(c):
            fire(c + 1, 1)
            drain_write(c, 0)
            fire(c + 2, 0)
            drain_write(c + 1, 1)

        fire(chunks - 1, 1)
        drain_write(chunks - 2, 0)
        drain_write(chunks - 1, 1)

    return gk(q2, table)


def _mm_body(x_ref, w_ref, b_ref, o_ref):
    x = x_ref[...].reshape(1024, LPAD * DIM // 128, 128).reshape(1024, LPAD * DIM)
    o_ref[...] = (
        lax.dot_general(
            x,
            w_ref[...],
            dimension_numbers=(((1,), (1,)), ((), ())),
            preferred_element_type=jnp.float32,
        )
        + b_ref[...]
    )


def _tc_matmul(x128, w_pad, b2):
    bm = 1024
    kp = LPAD * DIM
    return pl.pallas_call(
        _mm_body,
        grid=(BATCH // bm,),
        in_specs=[
            pl.BlockSpec((bm * kp // 128, 128), lambda i: (i, 0)),
            pl.BlockSpec((OUT, kp), lambda i: (0, 0)),
            pl.BlockSpec((1, OUT), lambda i: (0, 0)),
        ],
        out_specs=pl.BlockSpec((bm, OUT), lambda i: (i, 0)),
        out_shape=jax.ShapeDtypeStruct((BATCH, OUT), jnp.float32),
    )(x128, w_pad, b2)


def kernel(question, table, W, b):
    t2 = _tc_relayout(table.T).reshape(VPAD, DIM)
    qp = jnp.pad(_permute_idx(question), ((0, 0), (0, LPAD - QLEN)))
    q2 = qp.reshape(BLP // G, G)
    gathered = _sc_gather(q2, t2)
    x128 = gathered.reshape(BATCH * LPAD * DIM // 128, 128)
    w_pad = jnp.pad(W, ((0, 0), (0, LPAD * DIM - QLEN * DIM)))
    return _tc_matmul(x128, w_pad, b.reshape(1, OUT))


# speedup vs baseline: 3.0486x; 3.0486x over previous
import functools

import jax
import jax.numpy as jnp
from jax import lax
from jax.experimental import pallas as pl
from jax.experimental.pallas import tpu as pltpu
from jax.experimental.pallas import tpu_sc as plsc

NUM_EMB = 1000000
BATCH = 16384
QLEN = 50
LPAD = 56
DIM = 16
OUT = 128
BLP = BATCH * LPAD
G = 128
K = 8
CHUNK_ROWS = K * G
TBLK = 1024
WBLK = 8192
TGRID = -(-NUM_EMB // WBLK)
VPAD = TGRID * WBLK


def _relayout_body(x_ref, o_ref):
    x = x_ref[...]
    for j in range(WBLK // TBLK):
        xj = x[:, TBLK * j:TBLK * (j + 1)]
        xh = xj.reshape(DIM, 8, 128).swapaxes(0, 1).reshape(128, 128)
        o_ref[128 * j:128 * (j + 1), :] = xh.T


def _tc_relayout(t_t):
    return pl.pallas_call(
        _relayout_body,
        grid=(TGRID,),
        in_specs=[pl.BlockSpec((DIM, WBLK), lambda i: (0, i))],
        out_specs=pl.BlockSpec((WBLK // 8, 128), lambda i: (i, 0)),
        out_shape=jax.ShapeDtypeStruct((VPAD // 8, 128), jnp.float32),
    )(t_t)


def _permute_idx(v):
    return (v & ~(TBLK - 1)) + ((v & 127) << 3) + ((v >> 7) & 7)


def _sc_gather(q2, table):
    info = plsc.get_sparse_core_info()
    nc, ns = info.num_cores, info.num_subcores
    nw = nc * ns
    qrows_per_w = q2.shape[0] // nw
    chunks = qrows_per_w // K

    mesh = plsc.VectorSubcoreMesh(core_axis_name="c", subcore_axis_name="s")

    @functools.partial(
        pl.kernel,
        mesh=mesh,
        compiler_params=pltpu.CompilerParams(use_tc_tiling_on_sc=False),
        out_type=jax.ShapeDtypeStruct((BLP, DIM), jnp.float32),
        scratch_types=[
            pltpu.VMEM((2, K, G), jnp.int32),
            pltpu.VMEM((2, CHUNK_ROWS, DIM), jnp.float32),
            pltpu.SemaphoreType.DMA,
            pltpu.SemaphoreType.DMA,
        ],
    )
    def gk(q_hbm, t_hbm, out_hbm, idx_v, rows_v, sem0, sem1):
        sems = (sem0, sem1)
        wid = lax.axis_index("s") * nc + lax.axis_index("c")
        base = wid * qrows_per_w

        def fire(c, buf):
            pltpu.sync_copy(q_hbm.at[pl.ds(base + c * K, K)], idx_v.at[buf])
            for j in range(K):
                pltpu.async_copy(
                    t_hbm.at[idx_v.at[buf].at[j]],
                    rows_v.at[buf].at[pl.ds(j * G, G)],
                    sems[buf],
                )

        def drain_write(c, buf):
            pltpu.make_async_copy(
                t_hbm.at[pl.ds(0, CHUNK_ROWS)], rows_v.at[buf], sems[buf]
            ).wait()
            pltpu.sync_copy(
                rows_v.at[buf],
                out_hbm.at[pl.ds((base + c * K) * G, CHUNK_ROWS)],
            )

        fire(0, 0)

        @pl.loop(0, chunks - 2, step=2)
        def _pair(c):
            fire(c + 1, 1)
            drain_write(c, 0)
            fire(c + 2, 0)
            drain_write(c + 1, 1)

        fire(chunks - 1, 1)
        drain_write(chunks - 2, 0)
        drain_write(chunks - 1, 1)

    return gk(q2, table)


def _mm_body(x_ref, w_ref, b_ref, o_ref):
    x = x_ref[...].reshape(1024, LPAD * DIM // 128, 128).reshape(1024, LPAD * DIM)
    o_ref[...] = (
        lax.dot_general(
            x,
            w_ref[...],
            dimension_numbers=(((1,), (1,)), ((), ())),
            preferred_element_type=jnp.float32,
        )
        + b_ref[...]
    )


def _tc_matmul(x128, w_pad, b2):
    bm = 1024
    kp = LPAD * DIM
    return pl.pallas_call(
        _mm_body,
        grid=(BATCH // bm,),
        in_specs=[
            pl.BlockSpec((bm * kp // 128, 128), lambda i: (i, 0)),
            pl.BlockSpec((OUT, kp), lambda i: (0, 0)),
            pl.BlockSpec((1, OUT), lambda i: (0, 0)),
        ],
        out_specs=pl.BlockSpec((bm, OUT), lambda i: (i, 0)),
        out_shape=jax.ShapeDtypeStruct((BATCH, OUT), jnp.float32),
    )(x128, w_pad, b2)


def kernel(question, table, W, b):
    t2 = _tc_relayout(table.T).reshape(VPAD, DIM)
    pq = _permute_idx(question)
    qp = jnp.concatenate([pq, pq[:, : LPAD - QLEN]], axis=1)
    q2 = qp.reshape(BLP // G, G)
    gathered = _sc_gather(q2, t2)
    x128 = gathered.reshape(BATCH * LPAD * DIM // 128, 128)
    w_pad = jnp.pad(W, ((0, 0), (0, LPAD * DIM - QLEN * DIM)))
    return _tc_matmul(x128, w_pad, b.reshape(1, OUT))
